# baseline (speedup 1.0000x reference)
import functools

import jax
import jax.numpy as jnp
from jax import lax
from jax.experimental import pallas as pl
from jax.experimental.pallas import tpu as pltpu
from jax.experimental.pallas import tpu_sc as plsc

_UNK = 22
_THRESH = 838861
_K0 = 0
_K1 = 42
_KS2 = _K0 ^ _K1 ^ 0x1BD11BDA
_ROT = ((13, 15, 26, 6), (17, 29, 16, 24))

_NC = 2
_NS = 16
_NW = _NC * _NS
_LANES = 16
_UNROLL = 4


def _threefry_bits(idx):
    x0 = jnp.zeros(idx.shape, jnp.int32)
    x1 = idx + jnp.int32(_K1)
    ks = (_K0, _K1, _KS2)
    for i in range(5):
        for d in _ROT[i % 2]:
            x0 = x0 + x1
            x1 = (x1 << jnp.int32(d)) | lax.shift_right_logical(
                x1, jnp.int32(32 - d)
            )
            x1 = x1 ^ x0
        x0 = x0 + jnp.int32(ks[(i + 1) % 3])
        x1 = x1 + jnp.int32(ks[(i + 2) % 3] + i + 1)
    return x0 ^ x1


def _make_sc_kernel(rows, cols):
    chunk = rows * cols // _NW
    per_row = cols // chunk
    vecs = chunk // _LANES
    mesh = plsc.VectorSubcoreMesh(core_axis_name="c", subcore_axis_name="s")

    @functools.partial(
        pl.kernel,
        mesh=mesh,
        out_type=jax.ShapeDtypeStruct((rows, cols), jnp.int32),
        scratch_types=[pltpu.VMEM((chunk,), jnp.int32)],
    )
    def sc_kernel(x_hbm, o_hbm, buf):
        wid = lax.axis_index("s") * _NC + lax.axis_index("c")
        row = wid // per_row
        col0 = (wid % per_row) * chunk
        pltpu.sync_copy(x_hbm.at[row, pl.ds(col0, chunk)], buf)
        base = row * jnp.int32(cols) + col0
        lanes = lax.iota(jnp.int32, _LANES)
        span = _LANES * _UNROLL

        def body(j, carry):
            for u in range(_UNROLL):
                off = j * span + u * _LANES
                bits = _threefry_bits(base + off + lanes)
                m = lax.shift_right_logical(bits, jnp.int32(9)) < jnp.int32(
                    _THRESH
                )
                buf[pl.ds(off, _LANES)] = jnp.where(
                    m, jnp.int32(_UNK), buf[pl.ds(off, _LANES)]
                )
            return carry

        lax.fori_loop(0, vecs // _UNROLL, body, jnp.int32(0))
        pltpu.sync_copy(buf, o_hbm.at[row, pl.ds(col0, chunk)])

    return sc_kernel


@jax.jit
def kernel(x):
    rows, cols = x.shape
    return _make_sc_kernel(rows, cols)(x)


# speedup vs baseline: 10.6759x; 10.6759x over previous
import numpy as np

import jax
import jax.numpy as jnp
from jax import lax
from jax.experimental import pallas as pl

_UNK = 22
_THRESH = 838861
_K0 = 0
_K1 = 42
_KS2 = _K0 ^ _K1 ^ 0x1BD11BDA
_ROT = ((13, 15, 26, 6), (17, 29, 16, 24))


def _rotl(v, d):
    return lax.shift_right_logical(v, jnp.int32(32 - d)) | (v << jnp.int32(d))


def _threefry_bits_prekeyed(x1):
    x0 = x1
    x1 = _rotl(x1, _ROT[0][0]) ^ x0
    for d in _ROT[0][1:]:
        x0 = x0 + x1
        x1 = _rotl(x1, d) ^ x0
    ks = (_K0, _K1, _KS2)
    x0 = x0 + jnp.int32(ks[1])
    x1 = x1 + jnp.int32(ks[2] + 1)
    for i in range(1, 5):
        for d in _ROT[i % 2]:
            x0 = x0 + x1
            x1 = _rotl(x1, d) ^ x0
        x0 = x0 + jnp.int32(ks[(i + 1) % 3])
        x1 = x1 + jnp.int32(ks[(i + 2) % 3] + i + 1)
    return x0 ^ x1


def _threefry_mask_body(x_ref, o_ref):
    x = x_ref[...]
    rows, cols = x.shape
    half = cols // 2
    ps = (2 * rows, half)
    s = lax.broadcasted_iota(jnp.int32, (2 * rows, 1), 0)
    row_off = (s & jnp.int32(rows - 1)) * jnp.int32(cols) + (
        lax.shift_right_logical(s, jnp.int32(rows.bit_length() - 1))
        * jnp.int32(half)
    ) + jnp.int32(_K1)
    l = lax.broadcasted_iota(jnp.int32, ps, 1)
    x1 = row_off + l
    bits = _threefry_bits_prekeyed(x1)
    m8 = lax.shift_right_logical(bits, jnp.int32(9)) < jnp.int32(_THRESH)
    mask = jnp.concatenate([m8[:rows, :], m8[rows:, :]], axis=1)
    o_ref[...] = jnp.where(mask, jnp.int32(_UNK), x)


@jax.jit
def kernel(x):
    return pl.pallas_call(
        _threefry_mask_body,
        out_shape=jax.ShapeDtypeStruct(x.shape, x.dtype),
    )(x)


# speedup vs baseline: 10.7537x; 1.0073x over previous
import numpy as np

import jax
import jax.numpy as jnp
from jax import lax
from jax.experimental import pallas as pl

_UNK = 22
_THRESH = 838861
_K0 = 0
_K1 = 42
_KS2 = _K0 ^ _K1 ^ 0x1BD11BDA
_ROT = ((13, 15, 26, 6), (17, 29, 16, 24))


def _rotl(v, d):
    return lax.shift_right_logical(v, jnp.int32(32 - d)) | (v << jnp.int32(d))


def _threefry_bits_prekeyed(x1):
    x0 = x1
    x1 = _rotl(x1, _ROT[0][0]) ^ x0
    for d in _ROT[0][1:]:
        x0 = x0 + x1
        x1 = _rotl(x1, d) ^ x0
    ks = (_K0, _K1, _KS2)
    x0 = x0 + jnp.int32(ks[1])
    x1 = x1 + jnp.int32(ks[2] + 1)
    for i in range(1, 5):
        for d in _ROT[i % 2]:
            x0 = x0 + x1
            x1 = _rotl(x1, d) ^ x0
        x0 = x0 + jnp.int32(ks[(i + 1) % 3])
        x1 = x1 + jnp.int32(ks[(i + 2) % 3] + i + 1)
    return x0 ^ x1


def _threefry_mask_body(total_cols, x_ref, o_ref):
    x = x_ref[...]
    rows, cols = x.shape
    half = cols // 2
    ps = (2 * rows, half)
    col0 = pl.program_id(0) * jnp.int32(cols)
    s = lax.broadcasted_iota(jnp.int32, (2 * rows, 1), 0)
    row_off = (s & jnp.int32(rows - 1)) * jnp.int32(total_cols) + (
        lax.shift_right_logical(s, jnp.int32(rows.bit_length() - 1))
        * jnp.int32(half)
    ) + (col0 + jnp.int32(_K1))
    l = lax.broadcasted_iota(jnp.int32, ps, 1)
    x1 = row_off + l
    bits = _threefry_bits_prekeyed(x1)
    m8 = lax.shift_right_logical(bits, jnp.int32(9)) < jnp.int32(_THRESH)
    mask = jnp.concatenate([m8[:rows, :], m8[rows:, :]], axis=1)
    o_ref[...] = jnp.where(mask, jnp.int32(_UNK), x)


_N_BLOCKS = 2


@jax.jit
def kernel(x):
    import functools

    rows, cols = x.shape
    bcols = cols // _N_BLOCKS
    spec = pl.BlockSpec((rows, bcols), lambda i: (0, i))
    return pl.pallas_call(
        functools.partial(_threefry_mask_body, cols),
        grid=(_N_BLOCKS,),
        in_specs=[spec],
        out_specs=spec,
        out_shape=jax.ShapeDtypeStruct(x.shape, x.dtype),
    )(x)
